# Initial kernel scaffold; baseline (speedup 1.0000x reference)
import jax
import jax.numpy as jnp
from jax.experimental import pallas as pl


def kernel(x, hyperedges, hyperedge_attrs, W1, b1, W2, b2):
    raise NotImplementedError("write your pallas kernel here")


# speedup vs baseline: 7.8713x; 7.8713x over previous
import functools

import jax
import jax.numpy as jnp
from jax import lax
from jax.experimental import pallas as pl
from jax.experimental.pallas import tpu as pltpu
from jax.experimental.pallas import tpu_sc as plsc

N = 10000
D = 128
E = 320000
NC, NS, L = 2, 16, 16
NW = NC * NS
CH = 128
NCHUNK = -(-E // (NW * CH))
EP = NW * CH * NCHUNK
ROWS_PER_SUB = 640
NP = NS * ROWS_PER_SUB
DUMMY = N

_mesh = plsc.VectorSubcoreMesh(
    core_axis_name="c", subcore_axis_name="s", num_cores=NC, num_subcores=NS)

_f32 = jnp.float32


def _sc_pass_body(with_hist, acc, *refs):
    if with_hist:
        (table, ixg, ixs, part_out, histg_out, hists_out,
         ixg_v, ixs_v, rowbuf, histg_v, hists_v) = refs
    else:
        (table, ixg, ixs, part_out, ixg_v, ixs_v, rowbuf) = refs
        histg_out = hists_out = histg_v = hists_v = None

    cid = lax.axis_index("c")
    sid = lax.axis_index("s")
    wid = sid * NC + cid

    zeros16 = jnp.zeros((L,), _f32)
    ones16 = jnp.ones((L,), _f32)

    @pl.loop(0, CH)
    def _(i):
        for t in range(D // L):
            rowbuf[i, pl.ds(t * L, L)] = zeros16

    if with_hist:
        @pl.loop(0, NP // L)
        def _(i):
            histg_v[pl.ds(i * L, L)] = zeros16
            hists_v[pl.ds(i * L, L)] = zeros16

    @pl.loop(0, ROWS_PER_SUB // CH)
    def _(k):
        pltpu.sync_copy(rowbuf, acc.at[pl.ds(sid * ROWS_PER_SUB + k * CH, CH)])

    plsc.subcore_barrier()

    @pl.loop(0, NCHUNK)
    def _(j):
        pltpu.sync_copy(ixg.at[wid, j], ixg_v.at[0])
        pltpu.sync_copy(ixs.at[wid, j], ixs_v.at[0])
        pltpu.sync_copy(table.at[ixg_v.at[0]], rowbuf)
        if with_hist:
            for t in range(CH // L):
                ivg = ixg_v[0, pl.ds(t * L, L)]
                plsc.addupdate_scatter(histg_v, [ivg], ones16)
                ivs = ixs_v[0, pl.ds(t * L, L)]
                plsc.addupdate_scatter(hists_v, [ivs], ones16)
        pltpu.sync_copy(rowbuf, acc.at[ixs_v.at[0]], add=True)

    plsc.subcore_barrier()

    pltpu.sync_copy(acc.at[pl.ds(sid * ROWS_PER_SUB, ROWS_PER_SUB)],
                    part_out.at[cid, pl.ds(sid * ROWS_PER_SUB, ROWS_PER_SUB)])
    if with_hist:
        pltpu.sync_copy(histg_v, histg_out.at[wid])
        pltpu.sync_copy(hists_v, hists_out.at[wid])


def _make_sc_pass(with_hist):
    out_type = [jax.ShapeDtypeStruct((NC, NP, D), _f32)]
    scratch = [
        pltpu.VMEM_SHARED((NP, D), _f32),
        pltpu.VMEM((1, CH), jnp.int32),
        pltpu.VMEM((1, CH), jnp.int32),
        pltpu.VMEM((CH, D), _f32),
    ]
    if with_hist:
        out_type += [jax.ShapeDtypeStruct((NW, NP), _f32),
                     jax.ShapeDtypeStruct((NW, NP), _f32)]
        scratch += [pltpu.VMEM((NP,), _f32), pltpu.VMEM((NP,), _f32)]

    def body(*refs):
        n_io = 4 + (2 if with_hist else 0)
        acc = refs[n_io]
        _sc_pass_body(with_hist, acc, *refs[:n_io], *refs[n_io + 1:])

    return pl.kernel(body, out_type=out_type, mesh=_mesh,
                     scratch_types=scratch,
                     compiler_params=pltpu.CompilerParams(
                         needs_layout_passes=False))


_sc_pass_hist = _make_sc_pass(True)
_sc_pass_plain = _make_sc_pass(False)


_BLK = 512


def _mm_body(x_ref, w_ref, o_ref):
    o_ref[...] = jnp.dot(x_ref[...], w_ref[...],
                         preferred_element_type=_f32)


def _matmul(x, w):
    return pl.pallas_call(
        _mm_body,
        grid=(NP // _BLK,),
        in_specs=[pl.BlockSpec((_BLK, D), lambda i: (i, 0)),
                  pl.BlockSpec((D, D), lambda i: (0, 0))],
        out_specs=pl.BlockSpec((_BLK, D), lambda i: (i, 0)),
        out_shape=jax.ShapeDtypeStruct((NP, D), _f32),
    )(x, w)


def _combine_edge_body(p_ref, hb_ref, hd_ref, ef_ref, dinv_ref):
    b = jnp.sum(hb_ref[...], axis=0)
    binv = jnp.where(b > 0, 1.0 / b, 0.0)
    d = jnp.sum(hd_ref[...], axis=0)
    dinv_ref[...] = jnp.where(d > 0, 1.0 / d, 0.0)
    ef_ref[...] = binv[:, None] * (p_ref[0] + p_ref[1])


def _combine_edge(part, hist_b, hist_d):
    return pl.pallas_call(
        _combine_edge_body,
        grid=(NP // _BLK,),
        in_specs=[pl.BlockSpec((NC, _BLK, D), lambda i: (0, i, 0)),
                  pl.BlockSpec((NW, _BLK), lambda i: (0, i)),
                  pl.BlockSpec((NW, _BLK), lambda i: (0, i))],
        out_specs=[pl.BlockSpec((_BLK, D), lambda i: (i, 0)),
                   pl.BlockSpec((_BLK,), lambda i: (i,))],
        out_shape=[jax.ShapeDtypeStruct((NP, D), _f32),
                   jax.ShapeDtypeStruct((NP,), _f32)],
    )(part, hist_b, hist_d)


def _combine_edge2_body(p_ref, binv_ref, ef_ref):
    ef_ref[...] = binv_ref[...][:, None] * (p_ref[0] + p_ref[1])


def _combine_edge2(part, binv):
    return pl.pallas_call(
        _combine_edge2_body,
        grid=(NP // _BLK,),
        in_specs=[pl.BlockSpec((NC, _BLK, D), lambda i: (0, i, 0)),
                  pl.BlockSpec((_BLK,), lambda i: (i,))],
        out_specs=pl.BlockSpec((_BLK, D), lambda i: (i, 0)),
        out_shape=jax.ShapeDtypeStruct((NP, D), _f32),
    )(part, binv)


def _combine_node_mm_body(p_ref, dinv_ref, b_ref, w_ref, o_ref):
    i = pl.program_id(0)
    h = dinv_ref[...][:, None] * (p_ref[0] + p_ref[1]) + b_ref[...]
    h = jnp.maximum(h, 0.0)
    rid = i * _BLK + lax.broadcasted_iota(jnp.int32, (_BLK, 1), 0)
    h = jnp.where(rid < N, h, 0.0)
    o_ref[...] = jnp.dot(h, w_ref[...], preferred_element_type=_f32)


def _combine_node_mm(part, dinv, b1, w2):
    return pl.pallas_call(
        _combine_node_mm_body,
        grid=(NP // _BLK,),
        in_specs=[pl.BlockSpec((NC, _BLK, D), lambda i: (0, i, 0)),
                  pl.BlockSpec((_BLK,), lambda i: (i,)),
                  pl.BlockSpec((1, D), lambda i: (0, 0)),
                  pl.BlockSpec((D, D), lambda i: (0, 0))],
        out_specs=pl.BlockSpec((_BLK, D), lambda i: (i, 0)),
        out_shape=jax.ShapeDtypeStruct((NP, D), _f32),
    )(part, dinv, b1.reshape(1, D), w2)


def _final_body(p_ref, dinv_ref, b_ref, o_ref):
    o_ref[...] = (dinv_ref[...][:, None] * (p_ref[0] + p_ref[1])
                  + b_ref[...])


def _final(part, dinv, b2):
    return pl.pallas_call(
        _final_body,
        grid=(NP // _BLK,),
        in_specs=[pl.BlockSpec((NC, _BLK, D), lambda i: (0, i, 0)),
                  pl.BlockSpec((_BLK,), lambda i: (i,)),
                  pl.BlockSpec((1, D), lambda i: (0, 0))],
        out_specs=pl.BlockSpec((_BLK, D), lambda i: (i, 0)),
        out_shape=jax.ShapeDtypeStruct((NP, D), _f32),
    )(part, dinv, b2.reshape(1, D))


def kernel(x, hyperedges, hyperedge_attrs, W1, b1, W2, b2):
    del hyperedge_attrs
    pad = EP - E
    ni = jnp.concatenate(
        [hyperedges[0].astype(jnp.int32),
         jnp.full((pad,), DUMMY, jnp.int32)]).reshape(NW, NCHUNK, CH)
    ei = jnp.concatenate(
        [hyperedges[1].astype(jnp.int32),
         jnp.full((pad,), DUMMY, jnp.int32)]).reshape(NW, NCHUNK, CH)
    xp = jnp.zeros((NP, D), _f32).at[:N].set(x)

    x1 = _matmul(xp, W1)
    epart, hist_d, hist_b = _sc_pass_hist(x1, ni, ei)
    ef, dinv = _combine_edge(epart, hist_b, hist_d)
    (npart,) = _sc_pass_plain(ef, ei, ni)
    x2 = _combine_node_mm(npart, dinv, b1, W2)

    (epart2,) = _sc_pass_plain(x2, ni, ei)
    ef2, _ = _combine_edge(epart2, hist_b, hist_d)
    (npart2,) = _sc_pass_plain(ef2, ei, ni)
    return _final(npart2, dinv, b2)[:N]
